# baseline (speedup 1.0000x reference)
import functools
import math

import jax
import jax.numpy as jnp
from jax import lax
from jax.experimental import pallas as pl
from jax.experimental.pallas import tpu as pltpu
from jax.experimental.pallas import tpu_sc as plsc

_D = 768
_SCALE = math.sqrt(float(_D))
_NC = 2
_NS = 16
_NW = _NC * _NS
_LANES = 16
_CHUNK = 48
_NBUF = 3


@functools.cache
def _emb_call(n_batch: int, n_time: int):
    b_per_w = (n_batch * n_time) // _NW
    n_main = b_per_w // _CHUNK
    tail = b_per_w - n_main * _CHUNK
    mesh = plsc.VectorSubcoreMesh(core_axis_name="c", subcore_axis_name="s")

    @functools.partial(
        pl.kernel,
        mesh=mesh,
        out_type=jax.ShapeDtypeStruct((n_batch, n_time, _D), jnp.float32),
        scratch_types=[
            pltpu.VMEM((b_per_w,), jnp.int32),
            pltpu.VMEM((_NBUF, _CHUNK, _D), jnp.float32),
            pltpu.SemaphoreType.DMA,
            pltpu.SemaphoreType.DMA,
        ],
    )
    def run(idx_hbm, table_hbm, out_hbm, idx_v, buf, gsem, ssem):
        wid = lax.axis_index("s") * _NC + lax.axis_index("c")
        row = wid * b_per_w // n_time
        col = pl.multiple_of(lax.rem(wid * b_per_w, n_time), 8)
        pltpu.sync_copy(idx_hbm.at[row, pl.ds(col, b_per_w)], idx_v)

        def gather(c, slot, size):
            off = pl.multiple_of(c * _CHUNK, 8)
            return pltpu.async_copy(
                table_hbm.at[idx_v.at[pl.ds(off, size)]],
                buf.at[slot, pl.ds(0, size)], gsem)

        def scatter(c, slot, size):
            off = pl.multiple_of(col + c * _CHUNK, 8)
            return pltpu.async_copy(
                buf.at[slot, pl.ds(0, size)],
                out_hbm.at[row, pl.ds(off, size)], ssem)

        def wait_gather(size):
            pltpu.make_async_copy(
                table_hbm.at[idx_v.at[pl.ds(0, size)]],
                buf.at[0, pl.ds(0, size)], gsem).wait()

        def wait_scatter(size):
            pltpu.make_async_copy(
                buf.at[0, pl.ds(0, size)],
                out_hbm.at[row, pl.ds(col, size)], ssem).wait()

        def scale(slot, size):
            bref = buf.at[slot]

            def srow(r, carry):
                for j in range(_D // _LANES):
                    sl = pl.ds(j * _LANES, _LANES)
                    bref[r, sl] = bref[r, sl] * _SCALE
                return carry

            lax.fori_loop(0, size, srow, 0)

        gather(0, 0, _CHUNK)

        def body(c, carry):
            slot = lax.rem(c, _NBUF)
            nxt = lax.rem(c + 1, _NBUF)

            @pl.when(c >= _NBUF - 1)
            def _drain_prev():
                wait_scatter(_CHUNK)

            @pl.when(c + 1 < n_main)
            def _prefetch():
                gather(c + 1, nxt, _CHUNK)

            wait_gather(_CHUNK)
            scale(slot, _CHUNK)
            scatter(c, slot, _CHUNK)
            return carry

        lax.fori_loop(0, n_main, body, 0)

        if tail:
            c = n_main
            slot = c % _NBUF
            gather(c, slot, tail)
            wait_gather(tail)
            scale(slot, tail)
            scatter(c, slot, tail)
            wait_scatter(tail)
        for _ in range(min(_NBUF - 1, n_main)):
            wait_scatter(_CHUNK)

    return run


@jax.jit
def kernel(input_ids, token_emb_weight):
    b, t = input_ids.shape
    return _emb_call(b, t)(input_ids.astype(jnp.int32), token_emb_weight)


# speedup vs baseline: 1.0093x; 1.0093x over previous
import functools
import math

import jax
import jax.numpy as jnp
from jax import lax
from jax.experimental import pallas as pl
from jax.experimental.pallas import tpu as pltpu
from jax.experimental.pallas import tpu_sc as plsc

_D = 768
_SCALE = math.sqrt(float(_D))
_NC = 2
_NS = 16
_NW = _NC * _NS
_LANES = 16
_CHUNK = 56
_NBUF = 3


@functools.cache
def _emb_call(n_batch: int, n_time: int):
    b_per_w = (n_batch * n_time) // _NW
    n_main = b_per_w // _CHUNK
    tail = b_per_w - n_main * _CHUNK
    mesh = plsc.VectorSubcoreMesh(core_axis_name="c", subcore_axis_name="s")

    @functools.partial(
        pl.kernel,
        mesh=mesh,
        out_type=jax.ShapeDtypeStruct((n_batch, n_time, _D), jnp.float32),
        scratch_types=[
            pltpu.VMEM((b_per_w,), jnp.int32),
            pltpu.VMEM((_NBUF, _CHUNK, _D), jnp.float32),
            pltpu.SemaphoreType.DMA,
            pltpu.SemaphoreType.DMA,
            pltpu.SemaphoreType.DMA,
        ],
    )
    def run(idx_hbm, table_hbm, out_hbm, idx_v, buf, gsem, ssem, tsem):
        wid = lax.axis_index("s") * _NC + lax.axis_index("c")
        row = wid * b_per_w // n_time
        col = pl.multiple_of(lax.rem(wid * b_per_w, n_time), 8)
        pltpu.sync_copy(idx_hbm.at[row, pl.ds(col, b_per_w)], idx_v)

        def gather(c, slot, size):
            off = pl.multiple_of(c * _CHUNK, 8)
            return pltpu.async_copy(
                table_hbm.at[idx_v.at[pl.ds(off, size)]],
                buf.at[slot, pl.ds(0, size)], gsem)

        def scatter(c, slot, size):
            off = pl.multiple_of(col + c * _CHUNK, 8)
            return pltpu.async_copy(
                buf.at[slot, pl.ds(0, size)],
                out_hbm.at[row, pl.ds(off, size)], ssem)

        def wait_gather(size):
            pltpu.make_async_copy(
                table_hbm.at[idx_v.at[pl.ds(0, size)]],
                buf.at[0, pl.ds(0, size)], gsem).wait()

        def wait_scatter(size):
            pltpu.make_async_copy(
                buf.at[0, pl.ds(0, size)],
                out_hbm.at[row, pl.ds(col, size)], ssem).wait()

        def scale(slot, size):
            bref = buf.at[slot]

            def srow(r, carry):
                for j in range(_D // _LANES):
                    sl = pl.ds(j * _LANES, _LANES)
                    bref[r, sl] = bref[r, sl] * _SCALE
                return carry

            lax.fori_loop(0, size, srow, 0)

        t_off = pl.multiple_of(n_main * _CHUNK, 8)
        if tail:
            pltpu.async_copy(
                table_hbm.at[idx_v.at[pl.ds(t_off, tail)]],
                buf.at[0, pl.ds(0, tail)], tsem)
        gather(0, 1, _CHUNK)
        if tail:
            pltpu.make_async_copy(
                table_hbm.at[idx_v.at[pl.ds(0, tail)]],
                buf.at[0, pl.ds(0, tail)], tsem).wait()
            scale(0, tail)
            pltpu.async_copy(
                buf.at[0, pl.ds(0, tail)],
                out_hbm.at[row, pl.ds(col + t_off, tail)], tsem)

        def wait_tail_scatter():
            pltpu.make_async_copy(
                buf.at[0, pl.ds(0, tail)],
                out_hbm.at[row, pl.ds(col, tail)], tsem).wait()

        def body(c, carry):
            slot = lax.rem(c + 1, _NBUF)
            nxt = lax.rem(c + 2, _NBUF)

            @pl.when(c >= _NBUF - 1)
            def _drain_prev():
                wait_scatter(_CHUNK)

            if tail:
                @pl.when(c == _NBUF - 2)
                def _drain_tail():
                    wait_tail_scatter()

            @pl.when(c + 1 < n_main)
            def _prefetch():
                gather(c + 1, nxt, _CHUNK)

            wait_gather(_CHUNK)
            scale(slot, _CHUNK)
            scatter(c, slot, _CHUNK)
            return carry

        lax.fori_loop(0, n_main, body, 0)

        if tail and n_main < _NBUF - 1:
            wait_tail_scatter()
        for _ in range(min(_NBUF - 1, n_main)):
            wait_scatter(_CHUNK)

    return run


@jax.jit
def kernel(input_ids, token_emb_weight):
    b, t = input_ids.shape
    return _emb_call(b, t)(input_ids.astype(jnp.int32), token_emb_weight)
